# baseline (speedup 1.0000x reference)
import functools
import math

import jax
import jax.numpy as jnp
from jax import lax
from jax.experimental import pallas as pl
from jax.experimental.pallas import tpu as pltpu
from jax.experimental.pallas import tpu_sc as plsc

MODEL_DIM = 64
SCALE = math.sqrt(MODEL_DIM)

NC = 2
NS = 16
NW = NC * NS
LANES = 16

RG = 128
G = 2
UR = RG * G
SEQ = 200
BATCH = 4096
UNITS_PER_SEQ = BATCH // UR
N_UNITS = SEQ * UNITS_PER_SEQ
UPW = N_UNITS // NW


def _emb_body(x_hbm, table_hbm, out_hbm, idx_v, rows_v, phys_v,
              g0, g1, s0, s1, i0, i1):
  wid = lax.axis_index("s") * NC + lax.axis_index("c")
  u0 = wid * UPW

  gsem = (g0, g1)
  ssem = (s0, s1)
  isem = (i0, i1)

  def unit_su(u):
    uu = u0 + u
    s = uu // UNITS_PER_SEQ
    j = uu % UNITS_PER_SEQ
    return s, j

  def sync_idx(u, b):
    s, j = unit_su(u)
    pltpu.sync_copy(x_hbm.at[s, pl.ds(j * G, G)], idx_v.at[b])

  def start_idx(u, b):
    s, j = unit_su(u)
    s = jnp.minimum(s, SEQ - 1)
    pltpu.make_async_copy(
        x_hbm.at[s, pl.ds(j * G, G)], idx_v.at[b], isem[b]).start()

  def wait_idx(b):
    pltpu.make_async_copy(
        x_hbm.at[0, pl.ds(0, G)], idx_v.at[b], isem[b]).wait()

  def start_gather(b):
    for g in range(G):
      pltpu.make_async_copy(
          table_hbm.at[idx_v.at[b, g]],
          rows_v.at[b, pl.ds(g * RG, RG)],
          gsem[b]).start()

  def wait_gather(b):
    pltpu.make_async_copy(
        table_hbm.at[pl.ds(0, UR)], rows_v.at[b], gsem[b]).wait()

  iota = lax.iota(jnp.int32, LANES)
  rowvecs = [[(128 * c + 16 * li) + iota for li in range(8)]
             for c in range(G)]

  def transpose_scale(b):
    @pl.loop(0, 8)
    def _(d_hi):
      colbase = jnp.broadcast_to(8 * d_hi, (LANES,))
      for c in range(G):
        for r in range(8):
          colvec = colbase + r
          for li in range(8):
            v = plsc.load_gather(rows_v.at[b], [rowvecs[c][li], colvec])
            phys_v[b, d_hi, c, r, pl.ds(li * LANES, LANES)] = v * SCALE

  def start_store(u, b):
    s, j = unit_su(u)
    pltpu.make_async_copy(
        phys_v.at[b], out_hbm.at[s, :, pl.ds(j * G, G)], ssem[b]).start()

  def wait_store(b):
    pltpu.make_async_copy(
        phys_v.at[b], out_hbm.at[0, :, pl.ds(0, G)], ssem[b]).wait()

  sync_idx(0, 0)
  start_gather(0)
  sync_idx(1, 1)
  start_gather(1)
  wait_gather(0)
  start_idx(2, 0)
  transpose_scale(0)
  start_store(0, 0)

  @pl.loop(1, UPW - 1, step=2)
  def _(i):
    for b, off in ((1, 0), (0, 1)):
      u = i + off
      wait_store(1 - b)
      wait_idx(1 - b)
      start_gather(1 - b)
      wait_gather(b)
      start_idx(u + 2, b)
      transpose_scale(b)
      start_store(u, b)

  wait_gather(1)
  transpose_scale(1)
  start_store(UPW - 1, 1)
  wait_idx(0)
  wait_store(0)
  wait_store(1)


@jax.jit
def _emb_lookup(x, table):
  xv = jnp.swapaxes(x, 0, 1).reshape(SEQ, BATCH // RG, RG)

  kern = pl.kernel(
      _emb_body,
      out_type=jax.ShapeDtypeStruct((SEQ, 8, BATCH // RG, 8, RG),
                                    jnp.float32),
      mesh=plsc.VectorSubcoreMesh(core_axis_name="c", subcore_axis_name="s"),
      compiler_params=pltpu.CompilerParams(use_tc_tiling_on_sc=False,
                                           needs_layout_passes=False),
      scratch_types=[
          pltpu.VMEM((2, G, RG), jnp.int32),
          pltpu.VMEM((2, UR, MODEL_DIM), jnp.float32),
          pltpu.VMEM((2, 8, G, 8, RG), jnp.float32),
          pltpu.SemaphoreType.DMA,
          pltpu.SemaphoreType.DMA,
          pltpu.SemaphoreType.DMA,
          pltpu.SemaphoreType.DMA,
          pltpu.SemaphoreType.DMA,
          pltpu.SemaphoreType.DMA,
      ],
  )
  out5 = kern(xv, table)
  return out5.transpose(2, 4, 0, 1, 3).reshape(BATCH, SEQ, MODEL_DIM)


def kernel(x, table):
  return _emb_lookup(x.astype(jnp.int32), table)


# speedup vs baseline: 1.6907x; 1.6907x over previous
import functools
import math

import jax
import jax.numpy as jnp
from jax import lax
from jax.experimental import pallas as pl
from jax.experimental.pallas import tpu as pltpu
from jax.experimental.pallas import tpu_sc as plsc

MODEL_DIM = 64
SCALE = math.sqrt(MODEL_DIM)

NC = 2
NS = 16
NW = NC * NS
LANES = 16

RG = 128
G = 2
CHUNK = RG * G
NBUF = 4


def _emb_body(n_chunks, x_hbm, table_hbm, out_hbm, idx_v, rows_v, sems):
  wid = lax.axis_index("s") * NC + lax.axis_index("c")
  xrow0 = wid * (n_chunks * G)
  obase = wid * (n_chunks * CHUNK)

  gsem = sems[0:NBUF]
  ssem = sems[NBUF:2 * NBUF]
  isem = sems[2 * NBUF:3 * NBUF]
  n_idx_rows = n_chunks * G * NW

  def sync_idx(c, b):
    pltpu.sync_copy(x_hbm.at[pl.ds(xrow0 + c * G, G)], idx_v.at[b])

  def start_idx(c, b):
    row = jnp.minimum(xrow0 + c * G, n_idx_rows - G)
    pltpu.make_async_copy(x_hbm.at[pl.ds(row, G)], idx_v.at[b],
                          isem[b]).start()

  def wait_idx(b):
    pltpu.make_async_copy(x_hbm.at[pl.ds(0, G)], idx_v.at[b],
                          isem[b]).wait()

  def start_gather(b):
    for g in range(G):
      pltpu.make_async_copy(
          table_hbm.at[idx_v.at[b, g]],
          rows_v.at[b, pl.ds(g * RG, RG)],
          gsem[b]).start()

  def wait_gather(b):
    pltpu.make_async_copy(
        table_hbm.at[pl.ds(0, CHUNK)], rows_v.at[b], gsem[b]).wait()

  def scale(b):
    @pl.loop(0, CHUNK, unroll=4)
    def _(j):
      for k in range(MODEL_DIM // LANES):
        v = rows_v[b, j, pl.ds(k * LANES, LANES)]
        rows_v[b, j, pl.ds(k * LANES, LANES)] = v * SCALE

  def start_store(c, b):
    pltpu.make_async_copy(
        rows_v.at[b], out_hbm.at[pl.ds(obase + c * CHUNK, CHUNK)],
        ssem[b]).start()

  def wait_store(b):
    pltpu.make_async_copy(
        rows_v.at[b], out_hbm.at[pl.ds(obase, CHUNK)], ssem[b]).wait()

  for c in range(NBUF):
    sync_idx(c, c)
  start_gather(0)
  start_gather(1)

  def steady(c, b, *, ws=True, wi=False, g2=True, i4=True):
    if ws:
      wait_store((b + 2) % NBUF)
    if g2:
      if wi:
        wait_idx((b + 2) % NBUF)
      start_gather((b + 2) % NBUF)
    wait_gather(b)
    if i4:
      start_idx(c + NBUF, b)
    scale(b)
    start_store(c, b)

  steady(0, 0, ws=False, wi=False)
  steady(1, 1, ws=False, wi=False)

  @pl.loop(2, n_chunks - 6, step=NBUF)
  def _(i):
    for off in range(NBUF):
      steady(i + off, (2 + off) % NBUF, wi=True)

  steady(n_chunks - 6, (n_chunks - 6) % NBUF, wi=True)
  steady(n_chunks - 5, (n_chunks - 5) % NBUF, wi=True)
  steady(n_chunks - 4, (n_chunks - 4) % NBUF, wi=True, i4=False)
  steady(n_chunks - 3, (n_chunks - 3) % NBUF, wi=True, i4=False)
  steady(n_chunks - 2, (n_chunks - 2) % NBUF, g2=False, i4=False)
  steady(n_chunks - 1, (n_chunks - 1) % NBUF, g2=False, i4=False)

  wait_store((n_chunks - 2) % NBUF)
  wait_store((n_chunks - 1) % NBUF)


@jax.jit
def _emb_lookup(x2d, table):
  B = x2d.shape[0] * x2d.shape[1]
  n_chunks = B // (NW * CHUNK)
  xv = x2d.reshape(B // RG, RG)

  kern = pl.kernel(
      functools.partial(_emb_body, n_chunks),
      out_type=jax.ShapeDtypeStruct((B, MODEL_DIM), jnp.float32),
      mesh=plsc.VectorSubcoreMesh(core_axis_name="c", subcore_axis_name="s"),
      compiler_params=pltpu.CompilerParams(use_tc_tiling_on_sc=False),
      scratch_types=[
          pltpu.VMEM((NBUF, G, RG), jnp.int32),
          pltpu.VMEM((NBUF, CHUNK, MODEL_DIM), jnp.float32),
          [pltpu.SemaphoreType.DMA] * (3 * NBUF),
      ],
  )
  return kern(xv, table)


def kernel(x, table):
  out = _emb_lookup(x.astype(jnp.int32), table)
  return out.reshape(x.shape[0], x.shape[1], MODEL_DIM)
